# baseline (speedup 1.0000x reference)
import functools

import jax
import jax.numpy as jnp
from jax import lax
from jax.experimental import pallas as pl
from jax.experimental.pallas import tpu as pltpu
from jax.experimental.pallas import tpu_sc as plsc

_N = 10000
_E = 320000
_D = 128
_NCLS = 64
_K = 4

_NP = 10240
_NC = 2
_NS = 16
_C = 64
_NCH = 160
_EP = _NC * _NS * _NCH * _C
_RPT = _NP // _NS

_sc_mesh = plsc.VectorSubcoreMesh(core_axis_name="c", subcore_axis_name="s")


@functools.partial(
    pl.kernel,
    out_type=jax.ShapeDtypeStruct((_NC, _NP), jnp.float32),
    mesh=_sc_mesh,
    scratch_types=[
        pltpu.VMEM_SHARED((_NP,), jnp.float32),
        pltpu.VMEM((_NCH, _C), jnp.int32),
        pltpu.VMEM((_C,), jnp.float32),
        pltpu.VMEM((_RPT,), jnp.float32),
    ],
)
def _deg_kernel(dst_hbm, out_hbm, acc, idx, ones, zbuf):
    c = lax.axis_index("c")
    s = lax.axis_index("s")
    for j in range(_C // 16):
        ones[pl.ds(16 * j, 16)] = jnp.ones((16,), jnp.float32)
    for j in range(_RPT // 16):
        zbuf[pl.ds(16 * j, 16)] = jnp.zeros((16,), jnp.float32)

    wid = c * _NS + s
    pltpu.sync_copy(dst_hbm.at[wid], idx)
    pltpu.sync_copy(zbuf, acc.at[pl.ds(s * _RPT, _RPT)])
    plsc.subcore_barrier()

    def body(i, carry):
        pltpu.sync_copy(ones, acc.at[idx.at[i]], add=True)
        return carry

    lax.fori_loop(0, _NCH, body, 0)
    plsc.subcore_barrier()
    pltpu.sync_copy(acc.at[pl.ds(s * _RPT, _RPT)],
                    out_hbm.at[c, pl.ds(s * _RPT, _RPT)])


@functools.partial(
    pl.kernel,
    out_type=jax.ShapeDtypeStruct((_NC, _NP, _D), jnp.float32),
    mesh=_sc_mesh,
    scratch_types=[
        pltpu.VMEM_SHARED((_NP, _D), jnp.float32),
        pltpu.VMEM((_NCH // 8, _C), jnp.int32),
        pltpu.VMEM((_NCH // 8, _C), jnp.int32),
        [pltpu.VMEM((_C, _D), jnp.float32) for _ in range(5)],
        [pltpu.SemaphoreType.DMA for _ in range(5)],
        [pltpu.SemaphoreType.DMA for _ in range(5)],
    ],
)
def _prop_kernel(z_hbm, src_hbm, dst_hbm, out_hbm, acc, sidx, didx, rows,
                 sems, ssems):
    c = lax.axis_index("c")
    s = lax.axis_index("s")
    for r in range(16):
        for j in range(_D // 16):
            rows[0][r, pl.ds(16 * j, 16)] = jnp.zeros((16,), jnp.float32)

    rbase = s * _RPT
    wid = c * _NS + s

    def zb(i, carry):
        pltpu.sync_copy(rows[0].at[pl.ds(0, 16), :],
                        acc.at[pl.ds(rbase + 16 * i, 16), :])
        return carry

    lax.fori_loop(0, _RPT // 16, zb, 0)
    plsc.subcore_barrier()

    nph = _NCH // 8
    for p in range(8):
        pltpu.sync_copy(src_hbm.at[wid, p], sidx)
        pltpu.sync_copy(dst_hbm.at[wid, p], didx)
        for t in range(4):
            pltpu.async_copy(z_hbm.at[sidx.at[t]], rows[t], sems[t])

        def body(j, carry):
            for t in range(5):
                m = 5 * j + t
                u = (t + 4) % 5
                pltpu.make_async_copy(z_hbm.at[sidx.at[0]], rows[t],
                                      sems[t]).wait()
                pltpu.async_copy(rows[t], acc.at[didx.at[m]], ssems[t],
                                 add=True)

                @pl.when((m >= 1) & (m + 4 < nph))
                def _():
                    pltpu.make_async_copy(rows[u], acc.at[didx.at[0]],
                                          ssems[u]).wait()

                @pl.when(m + 4 < nph)
                def _():
                    pltpu.async_copy(z_hbm.at[sidx.at[m + 4]], rows[u],
                                     sems[u])
            return carry

        lax.fori_loop(0, nph // 5, body, 0)
        for t in range(5):
            pltpu.make_async_copy(rows[t], acc.at[didx.at[0]],
                                  ssems[t]).wait()
    plsc.subcore_barrier()
    pltpu.sync_copy(acc.at[pl.ds(rbase, _RPT), :],
                    out_hbm.at[c, pl.ds(rbase, _RPT), :])


_B = 512


def _scale_body(dp_ref, f_ref, z_ref, n_ref, n2_ref):
    deg = jnp.maximum(dp_ref[0] + dp_ref[1], 1.0)
    n = lax.rsqrt(deg)
    n_ref[...] = n
    n2_ref[...] = n * n
    z_ref[...] = f_ref[...] * n


def _fin_body(p_ref, n2_ref, z_ref):
    z_ref[...] = n2_ref[...] * (p_ref[0] + p_ref[1])


def _mlp_body(f_ref, n_ref, p1_ref, p2_ref, p3_ref, p4_ref,
              w1_ref, b1_ref, w2_ref, b2_ref, o_ref):
    ssum = (p1_ref[0] + p1_ref[1] + p2_ref[0] + p2_ref[1]
            + p3_ref[0] + p3_ref[1] + p4_ref[0] + p4_ref[1])
    y = (f_ref[...] + n_ref[...] * ssum) * (1.0 / (_K + 1))
    h = jnp.dot(y, w1_ref[...], preferred_element_type=jnp.float32)
    h = jnp.maximum(h + b1_ref[...], 0.0)
    lg = jnp.dot(h, w2_ref[...], preferred_element_type=jnp.float32)
    lg = lg + b2_ref[...]
    m = jnp.max(lg, axis=-1, keepdims=True)
    lg = lg - m
    o_ref[...] = lg - jnp.log(jnp.sum(jnp.exp(lg), axis=-1, keepdims=True))


def _col_spec():
    return pl.BlockSpec((_B, 1), lambda i: (i, 0))


def _row_spec():
    return pl.BlockSpec((_B, _D), lambda i: (i, 0))


def _p_spec():
    return pl.BlockSpec((_NC, _B, _D), lambda i: (0, i, 0))


def kernel(feats, edge_index, W1, b1, W2, b2):
    pad = _N + (jnp.arange(_EP - _E, dtype=jnp.int32) % (_NP - _N))
    src_flat = jnp.concatenate([edge_index[0], pad])
    dst_flat = jnp.concatenate([edge_index[1], pad])
    src = src_flat.reshape(_NC * _NS, 8, _NCH // 8, _C)
    dst = dst_flat.reshape(_NC * _NS, 8, _NCH // 8, _C)
    dst3 = dst_flat.reshape(_NC * _NS, _NCH, _C)
    feats_p = jnp.concatenate(
        [feats, jnp.zeros((_NP - _N, _D), jnp.float32)], axis=0)

    degp = _deg_kernel(dst3)

    grid = (_NP // _B,)
    scale = pl.pallas_call(
        _scale_body,
        grid=grid,
        in_specs=[pl.BlockSpec((_NC, _B, 1), lambda i: (0, i, 0)),
                  _row_spec()],
        out_specs=[_row_spec(), _col_spec(), _col_spec()],
        out_shape=[jax.ShapeDtypeStruct((_NP, _D), jnp.float32),
                   jax.ShapeDtypeStruct((_NP, 1), jnp.float32),
                   jax.ShapeDtypeStruct((_NP, 1), jnp.float32)],
    )
    fin = pl.pallas_call(
        _fin_body,
        grid=grid,
        in_specs=[_p_spec(), _col_spec()],
        out_specs=_row_spec(),
        out_shape=jax.ShapeDtypeStruct((_NP, _D), jnp.float32),
    )

    z, norm_col, norm2_col = scale(degp[:, :, None], feats_p)
    parts = []
    for k in range(_K):
        p = _prop_kernel(z, src, dst)
        parts.append(p)
        if k < _K - 1:
            z = fin(p, norm2_col)

    mlp = pl.pallas_call(
        _mlp_body,
        grid=grid,
        in_specs=[
            _row_spec(), _col_spec(),
            _p_spec(), _p_spec(), _p_spec(), _p_spec(),
            pl.BlockSpec((_D, _D), lambda i: (0, 0)),
            pl.BlockSpec((1, _D), lambda i: (0, 0)),
            pl.BlockSpec((_D, _NCLS), lambda i: (0, 0)),
            pl.BlockSpec((1, _NCLS), lambda i: (0, 0)),
        ],
        out_specs=pl.BlockSpec((_B, _NCLS), lambda i: (i, 0)),
        out_shape=jax.ShapeDtypeStruct((_NP, _NCLS), jnp.float32),
    )
    out = mlp(feats_p, norm_col, parts[0], parts[1], parts[2], parts[3],
              W1.T, b1[None, :], W2.T, b2[None, :])
    return out[:_N]


# speedup vs baseline: 1.1042x; 1.1042x over previous
import functools

import jax
import jax.numpy as jnp
from jax import lax
from jax.experimental import pallas as pl
from jax.experimental.pallas import tpu as pltpu
from jax.experimental.pallas import tpu_sc as plsc

_N = 10000
_E = 320000
_D = 128
_NCLS = 64
_K = 4

_NP = 10240
_NC = 2
_NS = 16
_C = 64
_NCH = 160
_EP = _NC * _NS * _NCH * _C
_RPT = _NP // _NS

_sc_mesh = plsc.VectorSubcoreMesh(core_axis_name="c", subcore_axis_name="s")


@functools.partial(
    pl.kernel,
    out_type=jax.ShapeDtypeStruct((_NC, _NP), jnp.float32),
    mesh=_sc_mesh,
    scratch_types=[
        pltpu.VMEM_SHARED((_NP,), jnp.float32),
        pltpu.VMEM((_NCH, _C), jnp.int32),
        pltpu.VMEM((_C,), jnp.float32),
        pltpu.VMEM((_RPT,), jnp.float32),
    ],
)
def _deg_kernel(dst_hbm, out_hbm, acc, idx, ones, zbuf):
    c = lax.axis_index("c")
    s = lax.axis_index("s")
    for j in range(_C // 16):
        ones[pl.ds(16 * j, 16)] = jnp.ones((16,), jnp.float32)
    for j in range(_RPT // 16):
        zbuf[pl.ds(16 * j, 16)] = jnp.zeros((16,), jnp.float32)

    wid = c * _NS + s
    pltpu.sync_copy(dst_hbm.at[wid], idx)
    pltpu.sync_copy(zbuf, acc.at[pl.ds(s * _RPT, _RPT)])
    plsc.subcore_barrier()

    def body(i, carry):
        pltpu.sync_copy(ones, acc.at[idx.at[i]], add=True)
        return carry

    lax.fori_loop(0, _NCH, body, 0)
    plsc.subcore_barrier()
    pltpu.sync_copy(acc.at[pl.ds(s * _RPT, _RPT)],
                    out_hbm.at[c, pl.ds(s * _RPT, _RPT)])


@functools.partial(
    pl.kernel,
    out_type=jax.ShapeDtypeStruct((_NC, _NP, _D), jnp.float32),
    mesh=_sc_mesh,
    scratch_types=[
        pltpu.VMEM_SHARED((_NP, _D), jnp.float32),
        pltpu.VMEM((_NCH // 4, _C), jnp.int32),
        pltpu.VMEM((_NCH // 4, _C), jnp.int32),
        [pltpu.VMEM((_C, _D), jnp.float32) for _ in range(4)],
        [pltpu.SemaphoreType.DMA for _ in range(4)],
    ],
)
def _prop_kernel(z_hbm, src_hbm, dst_hbm, out_hbm, acc, sidx, didx, rows,
                 sems):
    c = lax.axis_index("c")
    s = lax.axis_index("s")
    for r in range(16):
        for j in range(_D // 16):
            rows[0][r, pl.ds(16 * j, 16)] = jnp.zeros((16,), jnp.float32)

    rbase = s * _RPT
    wid = c * _NS + s

    def zb(i, carry):
        pltpu.sync_copy(rows[0].at[pl.ds(0, 16), :],
                        acc.at[pl.ds(rbase + 16 * i, 16), :])
        return carry

    lax.fori_loop(0, _RPT // 16, zb, 0)
    plsc.subcore_barrier()

    nph = _NCH // 4
    for p in range(4):
        pltpu.sync_copy(src_hbm.at[wid, p], sidx)
        pltpu.sync_copy(dst_hbm.at[wid, p], didx)
        for t in range(4):
            pltpu.async_copy(z_hbm.at[sidx.at[t]], rows[t], sems[t])

        def body(j, carry):
            for t in range(4):
                m = 4 * j + t
                pltpu.make_async_copy(z_hbm.at[sidx.at[m]], rows[t],
                                      sems[t]).wait()
                pltpu.sync_copy(rows[t], acc.at[didx.at[m]], add=True)

                @pl.when(m + 4 < nph)
                def _():
                    pltpu.async_copy(z_hbm.at[sidx.at[m + 4]], rows[t],
                                     sems[t])
            return carry

        lax.fori_loop(0, nph // 4, body, 0)
    plsc.subcore_barrier()
    pltpu.sync_copy(acc.at[pl.ds(rbase, _RPT), :],
                    out_hbm.at[c, pl.ds(rbase, _RPT), :])


_B = 512


def _scale_body(f_ref, n_ref, z_ref):
    z_ref[...] = f_ref[...] * n_ref[...]


def _fin_body(p_ref, n2_ref, z_ref):
    z_ref[...] = n2_ref[...] * (p_ref[0] + p_ref[1])


def _mlp_body(f_ref, n_ref, p1_ref, p2_ref, p3_ref, p4_ref,
              w1_ref, b1_ref, w2_ref, b2_ref, o_ref):
    ssum = (p1_ref[0] + p1_ref[1] + p2_ref[0] + p2_ref[1]
            + p3_ref[0] + p3_ref[1] + p4_ref[0] + p4_ref[1])
    y = (f_ref[...] + n_ref[...] * ssum) * (1.0 / (_K + 1))
    h = jnp.dot(y, w1_ref[...], preferred_element_type=jnp.float32)
    h = jnp.maximum(h + b1_ref[...], 0.0)
    lg = jnp.dot(h, w2_ref[...], preferred_element_type=jnp.float32)
    lg = lg + b2_ref[...]
    m = jnp.max(lg, axis=-1, keepdims=True)
    lg = lg - m
    o_ref[...] = lg - jnp.log(jnp.sum(jnp.exp(lg), axis=-1, keepdims=True))


def _col_spec():
    return pl.BlockSpec((_B, 1), lambda i: (i, 0))


def _row_spec():
    return pl.BlockSpec((_B, _D), lambda i: (i, 0))


def _p_spec():
    return pl.BlockSpec((_NC, _B, _D), lambda i: (0, i, 0))


def kernel(feats, edge_index, W1, b1, W2, b2):
    pad = _N + (jnp.arange(_EP - _E, dtype=jnp.int32) % (_NP - _N))
    src_flat = jnp.concatenate([edge_index[0], pad])
    dst_flat = jnp.concatenate([edge_index[1], pad])
    src = src_flat.reshape(_NC * _NS, 4, _NCH // 4, _C)
    dst = dst_flat.reshape(_NC * _NS, 4, _NCH // 4, _C)
    dst3 = dst_flat.reshape(_NC * _NS, _NCH, _C)
    feats_p = jnp.concatenate(
        [feats, jnp.zeros((_NP - _N, _D), jnp.float32)], axis=0)

    degp = _deg_kernel(dst3)
    deg = jnp.clip(degp[0] + degp[1], 1.0, None)
    norm_col = lax.rsqrt(deg)[:, None]
    norm2_col = norm_col * norm_col

    grid = (_NP // _B,)
    scale = pl.pallas_call(
        _scale_body,
        grid=grid,
        in_specs=[_row_spec(), _col_spec()],
        out_specs=_row_spec(),
        out_shape=jax.ShapeDtypeStruct((_NP, _D), jnp.float32),
    )
    fin = pl.pallas_call(
        _fin_body,
        grid=grid,
        in_specs=[_p_spec(), _col_spec()],
        out_specs=_row_spec(),
        out_shape=jax.ShapeDtypeStruct((_NP, _D), jnp.float32),
    )

    z = scale(feats_p, norm_col)
    parts = []
    for k in range(_K):
        p = _prop_kernel(z, src, dst)
        parts.append(p)
        if k < _K - 1:
            z = fin(p, norm2_col)

    mlp = pl.pallas_call(
        _mlp_body,
        grid=grid,
        in_specs=[
            _row_spec(), _col_spec(),
            _p_spec(), _p_spec(), _p_spec(), _p_spec(),
            pl.BlockSpec((_D, _D), lambda i: (0, 0)),
            pl.BlockSpec((1, _D), lambda i: (0, 0)),
            pl.BlockSpec((_D, _NCLS), lambda i: (0, 0)),
            pl.BlockSpec((1, _NCLS), lambda i: (0, 0)),
        ],
        out_specs=pl.BlockSpec((_B, _NCLS), lambda i: (i, 0)),
        out_shape=jax.ShapeDtypeStruct((_NP, _NCLS), jnp.float32),
    )
    out = mlp(feats_p, norm_col, parts[0], parts[1], parts[2], parts[3],
              W1.T, b1[None, :], W2.T, b2[None, :])
    return out[:_N]


# speedup vs baseline: 1.1554x; 1.0463x over previous
import functools

import jax
import jax.numpy as jnp
from jax import lax
from jax.experimental import pallas as pl
from jax.experimental.pallas import tpu as pltpu
from jax.experimental.pallas import tpu_sc as plsc

_N = 10000
_E = 320000
_D = 128
_NCLS = 64
_K = 4

_NP = 10240
_NC = 2
_NS = 16
_C = 64
_NCH = 160
_EP = _NC * _NS * _NCH * _C
_RPT = _NP // _NS

_sc_mesh = plsc.VectorSubcoreMesh(core_axis_name="c", subcore_axis_name="s")


@functools.partial(
    pl.kernel,
    out_type=jax.ShapeDtypeStruct((_NC, _NP), jnp.float32),
    mesh=_sc_mesh,
    scratch_types=[
        pltpu.VMEM_SHARED((_NP,), jnp.float32),
        pltpu.VMEM((_NCH, _C), jnp.int32),
        pltpu.VMEM((_C,), jnp.float32),
        pltpu.VMEM((_RPT,), jnp.float32),
    ],
)
def _deg_kernel(dst_hbm, out_hbm, acc, idx, ones, zbuf):
    c = lax.axis_index("c")
    s = lax.axis_index("s")
    for j in range(_C // 16):
        ones[pl.ds(16 * j, 16)] = jnp.ones((16,), jnp.float32)
    for j in range(_RPT // 16):
        zbuf[pl.ds(16 * j, 16)] = jnp.zeros((16,), jnp.float32)

    wid = c * _NS + s
    pltpu.sync_copy(dst_hbm.at[wid], idx)
    pltpu.sync_copy(zbuf, acc.at[pl.ds(s * _RPT, _RPT)])
    plsc.subcore_barrier()

    def body(i, carry):
        pltpu.sync_copy(ones, acc.at[idx.at[i]], add=True)
        return carry

    lax.fori_loop(0, _NCH, body, 0)
    plsc.subcore_barrier()
    pltpu.sync_copy(acc.at[pl.ds(s * _RPT, _RPT)],
                    out_hbm.at[c, pl.ds(s * _RPT, _RPT)])


@functools.partial(
    pl.kernel,
    out_type=jax.ShapeDtypeStruct((_NC, _NP, _D), jnp.float32),
    mesh=_sc_mesh,
    scratch_types=[
        pltpu.VMEM_SHARED((_NP, _D), jnp.float32),
        pltpu.VMEM((_NCH // 4, _C), jnp.int32),
        pltpu.VMEM((_NCH // 4, _C), jnp.int32),
        [pltpu.VMEM((_C, _D), jnp.float32) for _ in range(4)],
        [pltpu.SemaphoreType.DMA for _ in range(4)],
    ],
)
def _prop_kernel(z_hbm, src_hbm, dst_hbm, out_hbm, acc, sidx, didx, rows,
                 sems):
    c = lax.axis_index("c")
    s = lax.axis_index("s")
    for r in range(16):
        for j in range(_D // 16):
            rows[0][r, pl.ds(16 * j, 16)] = jnp.zeros((16,), jnp.float32)

    rbase = s * _RPT
    wid = c * _NS + s

    def zb(i, carry):
        pltpu.sync_copy(rows[0].at[pl.ds(0, 16), :],
                        acc.at[pl.ds(rbase + 16 * i, 16), :])
        return carry

    lax.fori_loop(0, _RPT // 16, zb, 0)
    plsc.subcore_barrier()

    nph = _NCH // 4
    for p in range(4):
        pltpu.sync_copy(src_hbm.at[wid, p], sidx)
        pltpu.sync_copy(dst_hbm.at[wid, p], didx)
        for t in range(4):
            pltpu.async_copy(z_hbm.at[sidx.at[t]], rows[t], sems[t])

        def body(j, carry):
            for t in range(4):
                m = 4 * j + t
                pltpu.make_async_copy(z_hbm.at[sidx.at[m]], rows[t],
                                      sems[t]).wait()
                pltpu.sync_copy(rows[t], acc.at[didx.at[m]], add=True)

                @pl.when(m + 4 < nph)
                def _():
                    pltpu.async_copy(z_hbm.at[sidx.at[m + 4]], rows[t],
                                     sems[t])
            return carry

        lax.fori_loop(0, nph // 4, body, 0)
    plsc.subcore_barrier()
    pltpu.sync_copy(acc.at[pl.ds(rbase, _RPT), :],
                    out_hbm.at[c, pl.ds(rbase, _RPT), :])


_B = 1024


def _scale_body(f_ref, n_ref, z_ref):
    z_ref[...] = f_ref[...] * n_ref[...]


def _fin_body(p_ref, n2_ref, z_ref):
    z_ref[...] = n2_ref[...] * (p_ref[0] + p_ref[1])


def _mlp_body(f_ref, n_ref, p1_ref, p2_ref, p3_ref, p4_ref,
              w1_ref, b1_ref, w2_ref, b2_ref, o_ref):
    ssum = (p1_ref[0] + p1_ref[1] + p2_ref[0] + p2_ref[1]
            + p3_ref[0] + p3_ref[1] + p4_ref[0] + p4_ref[1])
    y = (f_ref[...] + n_ref[...] * ssum) * (1.0 / (_K + 1))
    h = jnp.dot(y, w1_ref[...], preferred_element_type=jnp.float32)
    h = jnp.maximum(h + b1_ref[...], 0.0)
    lg = jnp.dot(h, w2_ref[...], preferred_element_type=jnp.float32)
    lg = lg + b2_ref[...]
    m = jnp.max(lg, axis=-1, keepdims=True)
    lg = lg - m
    o_ref[...] = lg - jnp.log(jnp.sum(jnp.exp(lg), axis=-1, keepdims=True))


def _col_spec():
    return pl.BlockSpec((_B, 1), lambda i: (i, 0))


def _row_spec():
    return pl.BlockSpec((_B, _D), lambda i: (i, 0))


def _p_spec():
    return pl.BlockSpec((_NC, _B, _D), lambda i: (0, i, 0))


def kernel(feats, edge_index, W1, b1, W2, b2):
    pad = _N + (jnp.arange(_EP - _E, dtype=jnp.int32) % (_NP - _N))
    src_flat = jnp.concatenate([edge_index[0], pad])
    dst_flat = jnp.concatenate([edge_index[1], pad])
    src = src_flat.reshape(_NC * _NS, 4, _NCH // 4, _C)
    dst = dst_flat.reshape(_NC * _NS, 4, _NCH // 4, _C)
    dst3 = dst_flat.reshape(_NC * _NS, _NCH, _C)
    feats_p = jnp.concatenate(
        [feats, jnp.zeros((_NP - _N, _D), jnp.float32)], axis=0)

    degp = _deg_kernel(dst3)
    deg = jnp.clip(degp[0] + degp[1], 1.0, None)
    norm_col = lax.rsqrt(deg)[:, None]
    norm2_col = norm_col * norm_col

    grid = (_NP // _B,)
    scale = pl.pallas_call(
        _scale_body,
        grid=grid,
        in_specs=[_row_spec(), _col_spec()],
        out_specs=_row_spec(),
        out_shape=jax.ShapeDtypeStruct((_NP, _D), jnp.float32),
    )
    fin = pl.pallas_call(
        _fin_body,
        grid=grid,
        in_specs=[_p_spec(), _col_spec()],
        out_specs=_row_spec(),
        out_shape=jax.ShapeDtypeStruct((_NP, _D), jnp.float32),
    )

    z = scale(feats_p, norm_col)
    parts = []
    for k in range(_K):
        p = _prop_kernel(z, src, dst)
        parts.append(p)
        if k < _K - 1:
            z = fin(p, norm2_col)

    mlp = pl.pallas_call(
        _mlp_body,
        grid=grid,
        in_specs=[
            _row_spec(), _col_spec(),
            _p_spec(), _p_spec(), _p_spec(), _p_spec(),
            pl.BlockSpec((_D, _D), lambda i: (0, 0)),
            pl.BlockSpec((1, _D), lambda i: (0, 0)),
            pl.BlockSpec((_D, _NCLS), lambda i: (0, 0)),
            pl.BlockSpec((1, _NCLS), lambda i: (0, 0)),
        ],
        out_specs=pl.BlockSpec((_B, _NCLS), lambda i: (i, 0)),
        out_shape=jax.ShapeDtypeStruct((_NP, _NCLS), jnp.float32),
    )
    out = mlp(feats_p, norm_col, parts[0], parts[1], parts[2], parts[3],
              W1.T, b1[None, :], W2.T, b2[None, :])
    return out[:_N]


# speedup vs baseline: 1.1750x; 1.0170x over previous
import functools

import jax
import jax.numpy as jnp
from jax import lax
from jax.experimental import pallas as pl
from jax.experimental.pallas import tpu as pltpu
from jax.experimental.pallas import tpu_sc as plsc

_N = 10000
_E = 320000
_D = 128
_NCLS = 64
_K = 4

_NP = 10240
_NC = 2
_NS = 16
_C = 64
_NCH = 160
_EP = _NC * _NS * _NCH * _C
_RPT = _NP // _NS

_sc_mesh = plsc.VectorSubcoreMesh(core_axis_name="c", subcore_axis_name="s")


@functools.partial(
    pl.kernel,
    out_type=jax.ShapeDtypeStruct((_NC, _NP), jnp.float32),
    mesh=_sc_mesh,
    scratch_types=[
        pltpu.VMEM_SHARED((_NP,), jnp.float32),
        pltpu.VMEM((_NCH, _C), jnp.int32),
        pltpu.VMEM((_C,), jnp.float32),
        pltpu.VMEM((_RPT,), jnp.float32),
    ],
)
def _deg_kernel(dst_hbm, out_hbm, acc, idx, ones, zbuf):
    c = lax.axis_index("c")
    s = lax.axis_index("s")
    for j in range(_C // 16):
        ones[pl.ds(16 * j, 16)] = jnp.ones((16,), jnp.float32)
    for j in range(_RPT // 16):
        zbuf[pl.ds(16 * j, 16)] = jnp.zeros((16,), jnp.float32)

    wid = c * _NS + s
    pltpu.sync_copy(dst_hbm.at[wid], idx)
    pltpu.sync_copy(zbuf, acc.at[pl.ds(s * _RPT, _RPT)])
    plsc.subcore_barrier()

    def body(i, carry):
        pltpu.sync_copy(ones, acc.at[idx.at[i]], add=True)
        return carry

    lax.fori_loop(0, _NCH, body, 0)
    plsc.subcore_barrier()
    pltpu.sync_copy(acc.at[pl.ds(s * _RPT, _RPT)],
                    out_hbm.at[c, pl.ds(s * _RPT, _RPT)])


@functools.partial(
    pl.kernel,
    out_type=jax.ShapeDtypeStruct((_NC, _NP, _D), jnp.float32),
    mesh=_sc_mesh,
    scratch_types=[
        pltpu.VMEM_SHARED((_NP, _D), jnp.float32),
        pltpu.VMEM((_NCH // 4, _C), jnp.int32),
        pltpu.VMEM((_NCH // 4, _C), jnp.int32),
        [pltpu.VMEM((_C, _D), jnp.float32) for _ in range(4)],
        [pltpu.SemaphoreType.DMA for _ in range(4)],
    ],
)
def _prop_kernel(z_hbm, src_hbm, dst_hbm, out_hbm, acc, sidx, didx, rows,
                 sems):
    c = lax.axis_index("c")
    s = lax.axis_index("s")
    for r in range(16):
        for j in range(_D // 16):
            rows[0][r, pl.ds(16 * j, 16)] = jnp.zeros((16,), jnp.float32)

    rbase = s * _RPT
    wid = c * _NS + s

    def zb(i, carry):
        pltpu.sync_copy(rows[0].at[pl.ds(0, 16), :],
                        acc.at[pl.ds(rbase + 16 * i, 16), :])
        return carry

    lax.fori_loop(0, _RPT // 16, zb, 0)
    plsc.subcore_barrier()

    nph = _NCH // 4
    for p in range(4):
        pltpu.sync_copy(src_hbm.at[wid, p], sidx)
        pltpu.sync_copy(dst_hbm.at[wid, p], didx)
        for t in range(4):
            pltpu.async_copy(z_hbm.at[sidx.at[t]], rows[t], sems[t])

        def body(j, carry):
            for t in range(4):
                m = 4 * j + t
                pltpu.make_async_copy(z_hbm.at[sidx.at[m]], rows[t],
                                      sems[t]).wait()
                pltpu.sync_copy(rows[t], acc.at[didx.at[m]], add=True)

                @pl.when(m + 4 < nph)
                def _():
                    pltpu.async_copy(z_hbm.at[sidx.at[m + 4]], rows[t],
                                     sems[t])
            return carry

        lax.fori_loop(0, nph // 4, body, 0)
    plsc.subcore_barrier()
    pltpu.sync_copy(acc.at[pl.ds(rbase, _RPT), :],
                    out_hbm.at[c, pl.ds(rbase, _RPT), :])


_B = 2048


def _scale_body(f_ref, n_ref, z_ref):
    z_ref[...] = f_ref[...] * n_ref[...]


def _fin_body(p_ref, n2_ref, z_ref):
    z_ref[...] = n2_ref[...] * (p_ref[0] + p_ref[1])


def _mlp_body(f_ref, n_ref, p1_ref, p2_ref, p3_ref, p4_ref,
              w1_ref, b1_ref, w2_ref, b2_ref, o_ref):
    ssum = (p1_ref[0] + p1_ref[1] + p2_ref[0] + p2_ref[1]
            + p3_ref[0] + p3_ref[1] + p4_ref[0] + p4_ref[1])
    y = (f_ref[...] + n_ref[...] * ssum) * (1.0 / (_K + 1))
    h = jnp.dot(y, w1_ref[...], preferred_element_type=jnp.float32)
    h = jnp.maximum(h + b1_ref[...], 0.0)
    lg = jnp.dot(h, w2_ref[...], preferred_element_type=jnp.float32)
    lg = lg + b2_ref[...]
    m = jnp.max(lg, axis=-1, keepdims=True)
    lg = lg - m
    o_ref[...] = lg - jnp.log(jnp.sum(jnp.exp(lg), axis=-1, keepdims=True))


def _col_spec():
    return pl.BlockSpec((_B, 1), lambda i: (i, 0))


def _row_spec():
    return pl.BlockSpec((_B, _D), lambda i: (i, 0))


def _p_spec():
    return pl.BlockSpec((_NC, _B, _D), lambda i: (0, i, 0))


def kernel(feats, edge_index, W1, b1, W2, b2):
    pad = _N + (jnp.arange(_EP - _E, dtype=jnp.int32) % (_NP - _N))
    src_flat = jnp.concatenate([edge_index[0], pad])
    dst_flat = jnp.concatenate([edge_index[1], pad])
    src = src_flat.reshape(_NC * _NS, 4, _NCH // 4, _C)
    dst = dst_flat.reshape(_NC * _NS, 4, _NCH // 4, _C)
    dst3 = dst_flat.reshape(_NC * _NS, _NCH, _C)
    feats_p = jnp.concatenate(
        [feats, jnp.zeros((_NP - _N, _D), jnp.float32)], axis=0)

    degp = _deg_kernel(dst3)
    deg = jnp.clip(degp[0] + degp[1], 1.0, None)
    norm_col = lax.rsqrt(deg)[:, None]
    norm2_col = norm_col * norm_col

    grid = (_NP // _B,)
    scale = pl.pallas_call(
        _scale_body,
        grid=grid,
        in_specs=[_row_spec(), _col_spec()],
        out_specs=_row_spec(),
        out_shape=jax.ShapeDtypeStruct((_NP, _D), jnp.float32),
    )
    fin = pl.pallas_call(
        _fin_body,
        grid=grid,
        in_specs=[_p_spec(), _col_spec()],
        out_specs=_row_spec(),
        out_shape=jax.ShapeDtypeStruct((_NP, _D), jnp.float32),
    )

    z = scale(feats_p, norm_col)
    parts = []
    for k in range(_K):
        p = _prop_kernel(z, src, dst)
        parts.append(p)
        if k < _K - 1:
            z = fin(p, norm2_col)

    mlp = pl.pallas_call(
        _mlp_body,
        grid=grid,
        in_specs=[
            _row_spec(), _col_spec(),
            _p_spec(), _p_spec(), _p_spec(), _p_spec(),
            pl.BlockSpec((_D, _D), lambda i: (0, 0)),
            pl.BlockSpec((1, _D), lambda i: (0, 0)),
            pl.BlockSpec((_D, _NCLS), lambda i: (0, 0)),
            pl.BlockSpec((1, _NCLS), lambda i: (0, 0)),
        ],
        out_specs=pl.BlockSpec((_B, _NCLS), lambda i: (i, 0)),
        out_shape=jax.ShapeDtypeStruct((_NP, _NCLS), jnp.float32),
    )
    out = mlp(feats_p, norm_col, parts[0], parts[1], parts[2], parts[3],
              W1.T, b1[None, :], W2.T, b2[None, :])
    return out[:_N]
